# baseline (speedup 1.0000x reference)
import functools

import jax
import jax.numpy as jnp
from jax import lax
from jax.experimental import pallas as pl
from jax.experimental.pallas import tpu as pltpu
from jax.experimental.pallas import tpu_sc as plsc

N = 10000
E = 320000
D = 128
H = 128
NEG = 0.05

NC, NS, L = 2, 16, 16
NW = NC * NS
CB = 80
ROWS = E // CB
RPW = ROWS // NW
NP = 640
NPT = N // NS
SEG = 25


def _prep_body(x_ref, w_ref, wa_ref, b_ref, ht_ref, s_ref):
    xb = x_ref[...]
    ht = lax.dot_general(xb, w_ref[...], (((1,), (1,)), ((), ())),
                         preferred_element_type=jnp.float32)
    ht_ref[...] = ht
    s = lax.dot_general(ht, wa_ref[...], (((1,), (1,)), ((), ())),
                        preferred_element_type=jnp.float32)
    s_ref[...] = s + b_ref[...]


_BN = 1000


def _prep_call(x, w, wa2, b2):
    return pl.pallas_call(
        _prep_body,
        grid=(N // _BN,),
        in_specs=[
            pl.BlockSpec((_BN, D), lambda i: (i, 0)),
            pl.BlockSpec((H, D), lambda i: (0, 0)),
            pl.BlockSpec((2, H), lambda i: (0, 0)),
            pl.BlockSpec((1, 2), lambda i: (0, 0)),
        ],
        out_specs=[
            pl.BlockSpec((_BN, H), lambda i: (i, 0)),
            pl.BlockSpec((_BN, 2), lambda i: (i, 0)),
        ],
        out_shape=[
            jax.ShapeDtypeStruct((N, H), jnp.float32),
            jax.ShapeDtypeStruct((N, 2), jnp.float32),
        ],
    )(x, w, wa2, b2)


@functools.partial(
    pl.kernel,
    out_type=(
        jax.ShapeDtypeStruct((NW, RPW, CB), jnp.float32),
        jax.ShapeDtypeStruct((NC, NP, L), jnp.float32),
    ),
    mesh=plsc.VectorSubcoreMesh(core_axis_name="c", subcore_axis_name="s"),
    scratch_types=[
        pltpu.VMEM((N,), jnp.float32),
        pltpu.VMEM((N,), jnp.float32),
        pltpu.VMEM((RPW, CB), jnp.int32),
        pltpu.VMEM((RPW, CB), jnp.int32),
        pltpu.VMEM((RPW, CB), jnp.float32),
        pltpu.VMEM((NP, L), jnp.float32),
        pltpu.VMEM((NP // 128, 128), jnp.int32),
        pltpu.VMEM_SHARED((NP, L), jnp.float32),
    ],
    compiler_params=pltpu.CompilerParams(needs_layout_passes=False, use_tc_tiling_on_sc=False),
)
def _sc_edge(src_hbm, dst_hbm, s1_hbm, s2_hbm,
             h_out, hsum_out,
             s1_v, s2_v, src_v, dst_v, h_v, hsum_v, ridx_v, hsum_s):
    cid = lax.axis_index("c")
    sid = lax.axis_index("s")
    wid = sid * NC + cid

    for k in range(NP // 128):
        for c in range(8):
            ridx_v[k, pl.ds(c * 16, 16)] = (
                lax.iota(jnp.int32, 16) + (k * 128 + c * 16))

    def _z_hsum(r, carry):
        hsum_v[r, :] = jnp.zeros((16,), jnp.float32)
        return carry
    lax.fori_loop(0, NP, _z_hsum, 0)

    @pl.when(sid == 0)
    def _():
        pltpu.sync_copy(hsum_v, hsum_s)

    pltpu.sync_copy(s1_hbm, s1_v)
    pltpu.sync_copy(s2_hbm, s2_v)
    pltpu.sync_copy(src_hbm.at[wid], src_v)
    pltpu.sync_copy(dst_hbm.at[wid], dst_v)

    plsc.subcore_barrier()

    def _p1(r, c1):
        for c in range(CB // 16):
            srcg = src_v[r, pl.ds(c * 16, 16)]
            dstg = dst_v[r, pl.ds(c * 16, 16)]
            v = (plsc.load_gather(s1_v, [srcg])
                 + plsc.load_gather(s2_v, [dstg]))
            v = jnp.where(v >= 0.0, v, NEG * v)
            hg = jnp.exp(v)
            h_v[r, pl.ds(c * 16, 16)] = hg
            plsc.addupdate_scatter(hsum_v, [srcg >> 4, srcg & 15], hg)
        return c1
    lax.fori_loop(0, RPW, _p1, 0)

    pltpu.sync_copy(h_v, h_out.at[wid])

    for k in range(NP // 128):
        pltpu.sync_copy(hsum_v.at[pl.ds(k * 128, 128)],
                        hsum_s.at[ridx_v.at[k]], add=True)

    plsc.subcore_barrier()

    @pl.when(sid == 0)
    def _():
        pltpu.sync_copy(hsum_s, hsum_out.at[cid])


NB = 3


@functools.partial(
    pl.kernel,
    out_type=(
        jax.ShapeDtypeStruct((NC, NS, NPT, H), jnp.float32),
        jax.ShapeDtypeStruct((NW, RPW, CB), jnp.float32),
    ),
    mesh=plsc.VectorSubcoreMesh(core_axis_name="c", subcore_axis_name="s"),
    scratch_types=[
        pltpu.VMEM((SEG, CB), jnp.int32),
        pltpu.VMEM((SEG, CB), jnp.int32),
        pltpu.VMEM((SEG, CB), jnp.float32),
        pltpu.VMEM((NP * L // 128, 128), jnp.float32),
        [pltpu.VMEM((CB, H), jnp.float32)] * NB,
        [pltpu.SemaphoreType.DMA] * NB,
        [pltpu.SemaphoreType.DMA] * NB,
        pltpu.VMEM_SHARED((N, H), jnp.float32),
    ],
    compiler_params=pltpu.CompilerParams(needs_layout_passes=False, use_tc_tiling_on_sc=False),
)
def _sc_agg(src_hbm, dst_hbm, h_hbm, hsum_hbm, ht_hbm, acc_out, alpha_out,
            src_v, dst_v, h_v, hs_v, rbufs, gsems, ssems, acc_s):
    cid = lax.axis_index("c")
    sid = lax.axis_index("s")
    wid = sid * NC + cid

    pltpu.sync_copy(hsum_hbm.at[0], hs_v)
    pltpu.sync_copy(hsum_hbm.at[1], rbufs[0])

    def _add_hs(r, carry):
        for c in range(128 // 16):
            hs_v[r, pl.ds(c * 16, 16)] = (
                hs_v[r, pl.ds(c * 16, 16)] + rbufs[0][r, pl.ds(c * 16, 16)])
        return carry
    lax.fori_loop(0, NP * L // 128, _add_hs, 0)

    def _z_rows(r, carry):
        for c in range(H // 16):
            rbufs[0][r, pl.ds(c * 16, 16)] = jnp.zeros((16,), jnp.float32)
        return carry
    lax.fori_loop(0, CB, _z_rows, 0)

    for k in range(NPT // CB):
        pltpu.sync_copy(rbufs[0], acc_s.at[pl.ds(sid * NPT + k * CB, CB)])
    rem = NPT - (NPT // CB) * CB
    pltpu.sync_copy(rbufs[0].at[pl.ds(0, rem)],
                    acc_s.at[pl.ds(sid * NPT + (NPT // CB) * CB, rem)])

    plsc.subcore_barrier()

    npre = NB - 2

    def _seg(g, carry):
        pltpu.sync_copy(src_hbm.at[wid, pl.ds(g * SEG, SEG)], src_v)
        pltpu.sync_copy(dst_hbm.at[wid, pl.ds(g * SEG, SEG)], dst_v)
        pltpu.sync_copy(h_hbm.at[wid, pl.ds(g * SEG, SEG)], h_v)

        gdescs = {}
        sdescs = {}
        for rr in range(npre):
            gdescs[rr] = pltpu.async_copy(
                ht_hbm.at[dst_v.at[rr]], rbufs[rr % NB], gsems[rr % NB])

        for r in range(SEG):
            nxt = r + npre
            if nxt < SEG:
                if nxt >= NB:
                    sdescs[nxt - NB].wait()
                gdescs[nxt] = pltpu.async_copy(
                    ht_hbm.at[dst_v.at[nxt]], rbufs[nxt % NB], gsems[nxt % NB])
            gdescs[r].wait()
            buf = rbufs[r % NB]

            def _scale(j, r=r, buf=buf):
                hb = plsc.load_gather(h_v, [jnp.full((16,), r, jnp.int32),
                                            jnp.full((16,), j, jnp.int32)])
                for c in range(H // 16):
                    buf[j, pl.ds(c * 16, 16)] = (
                        buf[j, pl.ds(c * 16, 16)] * hb)
            plsc.parallel_loop(0, CB, unroll=4)(_scale)

            sdescs[r] = pltpu.async_copy(buf, acc_s.at[src_v.at[r]],
                                         ssems[r % NB], add=True)

        def _alpha(r, c4):
            for c in range(CB // 16):
                srcg = src_v[r, pl.ds(c * 16, 16)]
                hsg = plsc.load_gather(hs_v, [srcg >> 7, srcg & 127])
                h_v[r, pl.ds(c * 16, 16)] = h_v[r, pl.ds(c * 16, 16)] / hsg
            return c4
        lax.fori_loop(0, SEG, _alpha, 0)
        pltpu.sync_copy(h_v, alpha_out.at[wid, pl.ds(g * SEG, SEG)])

        for rr in range(max(0, SEG - NB), SEG):
            sdescs[rr].wait()
        return carry
    lax.fori_loop(0, RPW // SEG, _seg, 0)

    plsc.subcore_barrier()

    pltpu.sync_copy(acc_s.at[pl.ds(sid * NPT, NPT)],
                    acc_out.at[cid, sid])


def _fin_body(acc_ref, hs_ref, out_ref):
    a = acc_ref[0] + acc_ref[1]
    d = hs_ref[0] + hs_ref[1]
    out_ref[...] = jnp.where(d > 0.0, a / jnp.where(d > 0.0, d, 1.0), 0.0)


def _fin_call(acc_p, hs_r):
    return pl.pallas_call(
        _fin_body,
        grid=(N // _BN,),
        in_specs=[
            pl.BlockSpec((NC, _BN, H), lambda i: (0, i, 0)),
            pl.BlockSpec((NC, _BN, 1), lambda i: (0, i, 0)),
        ],
        out_specs=pl.BlockSpec((_BN, H), lambda i: (i, 0)),
        out_shape=jax.ShapeDtypeStruct((N, H), jnp.float32),
    )(acc_p, hs_r)


def kernel(x, edge_index, W_fc, W_a, b_a):
    src = edge_index[0].reshape(NW, RPW, CB)
    dst = edge_index[1].reshape(NW, RPW, CB)
    wa2 = W_a.reshape(2, H)
    b2 = jnp.concatenate([b_a, jnp.zeros((1,), jnp.float32)]).reshape(1, 2)

    ht, s = _prep_call(x, W_fc, wa2, b2)
    s1 = s[:, 0]
    s2 = s[:, 1]

    h2, hsum_p = _sc_edge(src, dst, s1, s2)
    acc_p, alpha2 = _sc_agg(src, dst, h2,
                            hsum_p.reshape(NC, NP * L // 128, 128), ht)

    hs_r = hsum_p.reshape(NC, NP * L)[:, :N].reshape(NC, N, 1)
    out = _fin_call(acc_p.reshape(NC, N, H), hs_r)
    return (out, alpha2.reshape(E))


# speedup vs baseline: 1.0456x; 1.0456x over previous
import functools

import jax
import jax.numpy as jnp
from jax import lax
from jax.experimental import pallas as pl
from jax.experimental.pallas import tpu as pltpu
from jax.experimental.pallas import tpu_sc as plsc

N = 10000
E = 320000
D = 128
H = 128
NEG = 0.05

NC, NS, L = 2, 16, 16
NW = NC * NS
CB = 80
ROWS = E // CB
RPW = ROWS // NW
NP = 640
NPT = N // NS
SEG = 25


def _prep_body(x_ref, w_ref, wa_ref, b_ref, ht_ref, s_ref):
    xb = x_ref[...]
    ht = lax.dot_general(xb, w_ref[...], (((1,), (1,)), ((), ())),
                         preferred_element_type=jnp.float32)
    ht_ref[...] = ht.astype(jnp.bfloat16)
    s = lax.dot_general(ht, wa_ref[...], (((1,), (1,)), ((), ())),
                        preferred_element_type=jnp.float32)
    s_ref[...] = s + b_ref[...]


_BN = 1000


def _prep_call(x, w, wa2, b2):
    return pl.pallas_call(
        _prep_body,
        grid=(N // _BN,),
        in_specs=[
            pl.BlockSpec((_BN, D), lambda i: (i, 0)),
            pl.BlockSpec((H, D), lambda i: (0, 0)),
            pl.BlockSpec((2, H), lambda i: (0, 0)),
            pl.BlockSpec((1, 2), lambda i: (0, 0)),
        ],
        out_specs=[
            pl.BlockSpec((_BN, H), lambda i: (i, 0)),
            pl.BlockSpec((_BN, 2), lambda i: (i, 0)),
        ],
        out_shape=[
            jax.ShapeDtypeStruct((N, H), jnp.bfloat16),
            jax.ShapeDtypeStruct((N, 2), jnp.float32),
        ],
    )(x, w, wa2, b2)


@functools.partial(
    pl.kernel,
    out_type=(
        jax.ShapeDtypeStruct((NW, RPW, CB), jnp.float32),
        jax.ShapeDtypeStruct((NC, NP, L), jnp.float32),
    ),
    mesh=plsc.VectorSubcoreMesh(core_axis_name="c", subcore_axis_name="s"),
    scratch_types=[
        pltpu.VMEM((N,), jnp.float32),
        pltpu.VMEM((N,), jnp.float32),
        pltpu.VMEM((RPW, CB), jnp.int32),
        pltpu.VMEM((RPW, CB), jnp.int32),
        pltpu.VMEM((RPW, CB), jnp.float32),
        pltpu.VMEM((NP, L), jnp.float32),
        pltpu.VMEM((NP // 128, 128), jnp.int32),
        pltpu.VMEM_SHARED((NP, L), jnp.float32),
    ],
    compiler_params=pltpu.CompilerParams(needs_layout_passes=False, use_tc_tiling_on_sc=False),
)
def _sc_edge(src_hbm, dst_hbm, s1_hbm, s2_hbm,
             h_out, hsum_out,
             s1_v, s2_v, src_v, dst_v, h_v, hsum_v, ridx_v, hsum_s):
    cid = lax.axis_index("c")
    sid = lax.axis_index("s")
    wid = sid * NC + cid

    for k in range(NP // 128):
        for c in range(8):
            ridx_v[k, pl.ds(c * 16, 16)] = (
                lax.iota(jnp.int32, 16) + (k * 128 + c * 16))

    def _z_hsum(r, carry):
        hsum_v[r, :] = jnp.zeros((16,), jnp.float32)
        return carry
    lax.fori_loop(0, NP, _z_hsum, 0)

    @pl.when(sid == 0)
    def _():
        pltpu.sync_copy(hsum_v, hsum_s)

    pltpu.sync_copy(s1_hbm, s1_v)
    pltpu.sync_copy(s2_hbm, s2_v)
    pltpu.sync_copy(src_hbm.at[wid], src_v)
    pltpu.sync_copy(dst_hbm.at[wid], dst_v)

    plsc.subcore_barrier()

    def _p1(r, c1):
        for c in range(CB // 16):
            srcg = src_v[r, pl.ds(c * 16, 16)]
            dstg = dst_v[r, pl.ds(c * 16, 16)]
            v = (plsc.load_gather(s1_v, [srcg])
                 + plsc.load_gather(s2_v, [dstg]))
            v = jnp.where(v >= 0.0, v, NEG * v)
            hg = jnp.exp(v)
            h_v[r, pl.ds(c * 16, 16)] = hg
            plsc.addupdate_scatter(hsum_v, [srcg >> 4, srcg & 15], hg)
        return c1
    lax.fori_loop(0, RPW, _p1, 0)

    pltpu.sync_copy(h_v, h_out.at[wid])

    for k in range(NP // 128):
        pltpu.sync_copy(hsum_v.at[pl.ds(k * 128, 128)],
                        hsum_s.at[ridx_v.at[k]], add=True)

    plsc.subcore_barrier()

    @pl.when(sid == 0)
    def _():
        pltpu.sync_copy(hsum_s, hsum_out.at[cid])


NB = 4
NFB = 2


@functools.partial(
    pl.kernel,
    out_type=jax.ShapeDtypeStruct((NC, NS, NPT, H), jnp.float32),
    mesh=plsc.VectorSubcoreMesh(core_axis_name="c", subcore_axis_name="s"),
    scratch_types=[
        pltpu.VMEM((SEG, CB), jnp.int32),
        pltpu.VMEM((SEG, CB), jnp.int32),
        pltpu.VMEM((SEG, CB), jnp.float32),
        [pltpu.VMEM((CB, H), jnp.bfloat16)] * NB,
        [pltpu.VMEM((CB, H), jnp.float32)] * NFB,
        [pltpu.SemaphoreType.DMA] * NB,
        [pltpu.SemaphoreType.DMA] * NFB,
        pltpu.VMEM_SHARED((N, H), jnp.float32),
    ],
    compiler_params=pltpu.CompilerParams(needs_layout_passes=False, use_tc_tiling_on_sc=False),
)
def _sc_agg(src_hbm, dst_hbm, h_hbm, ht_hbm, acc_out,
            src_v, dst_v, h_v, rbufs, fbufs, gsems, ssems, acc_s):
    cid = lax.axis_index("c")
    sid = lax.axis_index("s")
    wid = sid * NC + cid

    def _z_rows(r, carry):
        for c in range(H // 16):
            fbufs[0][r, pl.ds(c * 16, 16)] = jnp.zeros((16,), jnp.float32)
        return carry
    lax.fori_loop(0, CB, _z_rows, 0)

    for k in range(NPT // CB):
        pltpu.sync_copy(fbufs[0], acc_s.at[pl.ds(sid * NPT + k * CB, CB)])
    rem = NPT - (NPT // CB) * CB
    pltpu.sync_copy(fbufs[0].at[pl.ds(0, rem)],
                    acc_s.at[pl.ds(sid * NPT + (NPT // CB) * CB, rem)])

    plsc.subcore_barrier()

    npre = NB - 1
    ev2 = lax.iota(jnp.int32, 16) * 2

    def _seg(g, carry):
        pltpu.sync_copy(src_hbm.at[wid, pl.ds(g * SEG, SEG)], src_v)
        pltpu.sync_copy(dst_hbm.at[wid, pl.ds(g * SEG, SEG)], dst_v)
        pltpu.sync_copy(h_hbm.at[wid, pl.ds(g * SEG, SEG)], h_v)

        gdescs = {}
        sdescs = {}
        for rr in range(npre):
            gdescs[rr] = pltpu.async_copy(
                ht_hbm.at[dst_v.at[rr]], rbufs[rr % NB], gsems[rr % NB])

        for r in range(SEG):
            nxt = r + npre
            if nxt < SEG:
                gdescs[nxt] = pltpu.async_copy(
                    ht_hbm.at[dst_v.at[nxt]], rbufs[nxt % NB], gsems[nxt % NB])
            gdescs[r].wait()
            if r >= NFB:
                sdescs[r - NFB].wait()
            buf = rbufs[r % NB]
            fb = fbufs[r % NFB]

            def _scale(j, r=r, buf=buf, fb=fb):
                hb = plsc.load_gather(h_v, [jnp.full((16,), r, jnp.int32),
                                            jnp.full((16,), j, jnp.int32)])
                jv = jnp.full((16,), j, jnp.int32)
                for c in range(H // 32):
                    ab = buf[j, pl.ds(c * 32, 32)]
                    a, b = plsc.unpack(ab, format=plsc.PackFormat.INTERLEAVED)
                    plsc.store_scatter(fb, [jv, c * 32 + ev2], a * hb)
                    plsc.store_scatter(fb, [jv, c * 32 + 1 + ev2], b * hb)
            plsc.parallel_loop(0, CB, unroll=4)(_scale)

            sdescs[r] = pltpu.async_copy(fb, acc_s.at[src_v.at[r]],
                                         ssems[r % NFB], add=True)

        for rr in range(max(0, SEG - NFB), SEG):
            sdescs[rr].wait()
        return carry
    lax.fori_loop(0, RPW // SEG, _seg, 0)

    plsc.subcore_barrier()

    pltpu.sync_copy(acc_s.at[pl.ds(sid * NPT, NPT)],
                    acc_out.at[cid, sid])


def _fin_body(acc_ref, hs_ref, out_ref):
    a = acc_ref[0] + acc_ref[1]
    d = hs_ref[0] + hs_ref[1]
    out_ref[...] = jnp.where(d > 0.0, a / jnp.where(d > 0.0, d, 1.0), 0.0)


def _fin_call(acc_p, hs_r):
    return pl.pallas_call(
        _fin_body,
        grid=(N // _BN,),
        in_specs=[
            pl.BlockSpec((NC, _BN, H), lambda i: (0, i, 0)),
            pl.BlockSpec((NC, _BN, 1), lambda i: (0, i, 0)),
        ],
        out_specs=pl.BlockSpec((_BN, H), lambda i: (i, 0)),
        out_shape=jax.ShapeDtypeStruct((N, H), jnp.float32),
    )(acc_p, hs_r)


@functools.partial(
    pl.kernel,
    out_type=jax.ShapeDtypeStruct((NW, RPW, CB), jnp.float32),
    mesh=plsc.VectorSubcoreMesh(core_axis_name="c", subcore_axis_name="s"),
    scratch_types=[
        pltpu.VMEM((RPW, CB), jnp.float32),
        pltpu.VMEM((RPW, CB), jnp.int32),
        pltpu.VMEM((NP, L), jnp.float32),
        pltpu.VMEM((NP, L), jnp.float32),
    ],
    compiler_params=pltpu.CompilerParams(needs_layout_passes=False, use_tc_tiling_on_sc=False),
)
def _sc_alpha(h_hbm, src_hbm, hsum_hbm, alpha_out,
              h_v, src_v, hs0_v, hs1_v):
    cid = lax.axis_index("c")
    sid = lax.axis_index("s")
    wid = sid * NC + cid
    pltpu.sync_copy(h_hbm.at[wid], h_v)
    pltpu.sync_copy(src_hbm.at[wid], src_v)
    pltpu.sync_copy(hsum_hbm.at[0], hs0_v)
    pltpu.sync_copy(hsum_hbm.at[1], hs1_v)

    def _b(r, carry):
        for c in range(CB // 16):
            srcg = src_v[r, pl.ds(c * 16, 16)]
            hs = (plsc.load_gather(hs0_v, [srcg >> 4, srcg & 15])
                  + plsc.load_gather(hs1_v, [srcg >> 4, srcg & 15]))
            h_v[r, pl.ds(c * 16, 16)] = h_v[r, pl.ds(c * 16, 16)] / hs
        return carry
    lax.fori_loop(0, RPW, _b, 0)

    pltpu.sync_copy(h_v, alpha_out.at[wid])


def kernel(x, edge_index, W_fc, W_a, b_a):
    src = edge_index[0].reshape(NW, RPW, CB)
    dst = edge_index[1].reshape(NW, RPW, CB)
    wa2 = W_a.reshape(2, H)
    b2 = jnp.concatenate([b_a, jnp.zeros((1,), jnp.float32)]).reshape(1, 2)

    ht, s = _prep_call(x, W_fc, wa2, b2)
    s1 = s[:, 0]
    s2 = s[:, 1]

    h2, hsum_p = _sc_edge(src, dst, s1, s2)
    acc_p = _sc_agg(src, dst, h2, ht)
    alpha2 = _sc_alpha(h2, src, hsum_p)

    hs_r = hsum_p.reshape(NC, NP * L)[:, :N].reshape(NC, N, 1)
    out = _fin_call(acc_p.reshape(NC, N, H), hs_r)
    return (out, alpha2.reshape(E))


# speedup vs baseline: 1.1042x; 1.0561x over previous
import functools

import jax
import jax.numpy as jnp
from jax import lax
from jax.experimental import pallas as pl
from jax.experimental.pallas import tpu as pltpu
from jax.experimental.pallas import tpu_sc as plsc

N = 10000
E = 320000
D = 128
H = 128
NEG = 0.05

NC, NS, L = 2, 16, 16
NW = NC * NS
CB = 80
ROWS = E // CB
RPW = ROWS // NW
NP = 640
NPT = N // NS
SEG = 25


def _prep_body(x_ref, w_ref, wa_ref, b_ref, ht_ref, s_ref):
    xb = x_ref[...]
    ht = lax.dot_general(xb, w_ref[...], (((1,), (1,)), ((), ())),
                         preferred_element_type=jnp.float32)
    ht_ref[...] = ht.astype(jnp.bfloat16)
    s = lax.dot_general(ht, wa_ref[...], (((1,), (1,)), ((), ())),
                        preferred_element_type=jnp.float32)
    s_ref[...] = s + b_ref[...]


_BN = 1000


def _prep_call(x, w, wa2, b2):
    return pl.pallas_call(
        _prep_body,
        grid=(N // _BN,),
        in_specs=[
            pl.BlockSpec((_BN, D), lambda i: (i, 0)),
            pl.BlockSpec((H, D), lambda i: (0, 0)),
            pl.BlockSpec((2, H), lambda i: (0, 0)),
            pl.BlockSpec((1, 2), lambda i: (0, 0)),
        ],
        out_specs=[
            pl.BlockSpec((_BN, H), lambda i: (i, 0)),
            pl.BlockSpec((_BN, 2), lambda i: (i, 0)),
        ],
        out_shape=[
            jax.ShapeDtypeStruct((N, H), jnp.bfloat16),
            jax.ShapeDtypeStruct((N, 2), jnp.float32),
        ],
    )(x, w, wa2, b2)


@functools.partial(
    pl.kernel,
    out_type=(
        jax.ShapeDtypeStruct((NW, RPW, CB), jnp.float32),
        jax.ShapeDtypeStruct((NC, NP, L), jnp.float32),
    ),
    mesh=plsc.VectorSubcoreMesh(core_axis_name="c", subcore_axis_name="s"),
    scratch_types=[
        pltpu.VMEM((N,), jnp.float32),
        pltpu.VMEM((N,), jnp.float32),
        pltpu.VMEM((RPW, CB), jnp.int32),
        pltpu.VMEM((RPW, CB), jnp.int32),
        pltpu.VMEM((RPW, CB), jnp.float32),
        pltpu.VMEM((NP, L), jnp.float32),
        pltpu.VMEM((NP // 128, 128), jnp.int32),
        pltpu.VMEM_SHARED((NP, L), jnp.float32),
        [pltpu.SemaphoreType.DMA] * 4,
    ],
    compiler_params=pltpu.CompilerParams(needs_layout_passes=False, use_tc_tiling_on_sc=False),
)
def _sc_edge(src_hbm, dst_hbm, s1_hbm, s2_hbm,
             h_out, hsum_out,
             s1_v, s2_v, src_v, dst_v, h_v, hsum_v, ridx_v, hsum_s, lsems):
    cid = lax.axis_index("c")
    sid = lax.axis_index("s")
    wid = sid * NC + cid

    ld = [pltpu.async_copy(s1_hbm, s1_v, lsems[0]),
          pltpu.async_copy(s2_hbm, s2_v, lsems[1]),
          pltpu.async_copy(src_hbm.at[wid], src_v, lsems[2]),
          pltpu.async_copy(dst_hbm.at[wid], dst_v, lsems[3])]

    for k in range(NP // 128):
        for c in range(8):
            ridx_v[k, pl.ds(c * 16, 16)] = (
                lax.iota(jnp.int32, 16) + (k * 128 + c * 16))

    def _z_hsum(r, carry):
        hsum_v[r, :] = jnp.zeros((16,), jnp.float32)
        return carry
    lax.fori_loop(0, NP, _z_hsum, 0)

    @pl.when(sid == 0)
    def _():
        pltpu.sync_copy(hsum_v, hsum_s)

    for d in ld:
        d.wait()

    plsc.subcore_barrier()

    def _p1(r, c1):
        for c in range(CB // 16):
            srcg = src_v[r, pl.ds(c * 16, 16)]
            dstg = dst_v[r, pl.ds(c * 16, 16)]
            v = (plsc.load_gather(s1_v, [srcg])
                 + plsc.load_gather(s2_v, [dstg]))
            v = jnp.where(v >= 0.0, v, NEG * v)
            hg = jnp.exp(v)
            h_v[r, pl.ds(c * 16, 16)] = hg
            plsc.addupdate_scatter(hsum_v, [srcg >> 4, srcg & 15], hg)
        return c1
    lax.fori_loop(0, RPW, _p1, 0)

    pltpu.sync_copy(h_v, h_out.at[wid])

    for k in range(NP // 128):
        pltpu.sync_copy(hsum_v.at[pl.ds(k * 128, 128)],
                        hsum_s.at[ridx_v.at[k]], add=True)

    plsc.subcore_barrier()

    @pl.when(sid == 0)
    def _():
        pltpu.sync_copy(hsum_s, hsum_out.at[cid])


NB = 4
NFB = 2


@functools.partial(
    pl.kernel,
    out_type=jax.ShapeDtypeStruct((NC, NS, NPT, H), jnp.float32),
    mesh=plsc.VectorSubcoreMesh(core_axis_name="c", subcore_axis_name="s"),
    scratch_types=[
        pltpu.VMEM((SEG, CB), jnp.int32),
        pltpu.VMEM((SEG, CB), jnp.int32),
        pltpu.VMEM((SEG, CB), jnp.float32),
        [pltpu.VMEM((CB, H), jnp.bfloat16)] * NB,
        [pltpu.VMEM((CB, H), jnp.float32)] * NFB,
        [pltpu.SemaphoreType.DMA] * NB,
        [pltpu.SemaphoreType.DMA] * NFB,
        [pltpu.SemaphoreType.DMA] * 3,
        pltpu.VMEM_SHARED((N, H), jnp.float32),
    ],
    compiler_params=pltpu.CompilerParams(needs_layout_passes=False, use_tc_tiling_on_sc=False),
)
def _sc_agg(src_hbm, dst_hbm, h_hbm, ht_hbm, acc_out,
            src_v, dst_v, h_v, rbufs, fbufs, gsems, ssems, lsems, acc_s):
    cid = lax.axis_index("c")
    sid = lax.axis_index("s")
    wid = sid * NC + cid

    def _z_rows(r, carry):
        for c in range(H // 16):
            fbufs[0][r, pl.ds(c * 16, 16)] = jnp.zeros((16,), jnp.float32)
        return carry
    lax.fori_loop(0, CB, _z_rows, 0)

    for k in range(NPT // CB):
        pltpu.sync_copy(fbufs[0], acc_s.at[pl.ds(sid * NPT + k * CB, CB)])
    rem = NPT - (NPT // CB) * CB
    pltpu.sync_copy(fbufs[0].at[pl.ds(0, rem)],
                    acc_s.at[pl.ds(sid * NPT + (NPT // CB) * CB, rem)])

    plsc.subcore_barrier()

    npre = NB - 1
    ev2 = lax.iota(jnp.int32, 16) * 2

    def _seg(g, carry):
        lds = [pltpu.async_copy(src_hbm.at[wid, pl.ds(g * SEG, SEG)],
                                src_v, lsems[0]),
               pltpu.async_copy(dst_hbm.at[wid, pl.ds(g * SEG, SEG)],
                                dst_v, lsems[1]),
               pltpu.async_copy(h_hbm.at[wid, pl.ds(g * SEG, SEG)],
                                h_v, lsems[2])]
        for d in lds:
            d.wait()
        gdescs = {}
        sdescs = {}
        for rr in range(npre):
            gdescs[rr] = pltpu.async_copy(
                ht_hbm.at[dst_v.at[rr]], rbufs[rr % NB], gsems[rr % NB])

        for r in range(SEG):
            nxt = r + npre
            if nxt < SEG:
                gdescs[nxt] = pltpu.async_copy(
                    ht_hbm.at[dst_v.at[nxt]], rbufs[nxt % NB], gsems[nxt % NB])
            gdescs[r].wait()
            if r >= NFB:
                sdescs[r - NFB].wait()
            buf = rbufs[r % NB]
            fb = fbufs[r % NFB]

            def _scale(j, r=r, buf=buf, fb=fb):
                hb = plsc.load_gather(h_v, [jnp.full((16,), r, jnp.int32),
                                            jnp.full((16,), j, jnp.int32)])
                jv = jnp.full((16,), j, jnp.int32)
                for c in range(H // 32):
                    ab = buf[j, pl.ds(c * 32, 32)]
                    a, b = plsc.unpack(ab, format=plsc.PackFormat.INTERLEAVED)
                    plsc.store_scatter(fb, [jv, c * 32 + ev2], a * hb)
                    plsc.store_scatter(fb, [jv, c * 32 + 1 + ev2], b * hb)
            plsc.parallel_loop(0, CB, unroll=4)(_scale)

            sdescs[r] = pltpu.async_copy(fb, acc_s.at[src_v.at[r]],
                                         ssems[r % NFB], add=True)

        for rr in range(max(0, SEG - NFB), SEG):
            sdescs[rr].wait()
        return carry
    lax.fori_loop(0, RPW // SEG, _seg, 0)

    plsc.subcore_barrier()

    pltpu.sync_copy(acc_s.at[pl.ds(sid * NPT, NPT)],
                    acc_out.at[cid, sid])


def _fin_body(acc_ref, hs_ref, out_ref):
    a = acc_ref[0] + acc_ref[1]
    d = hs_ref[0] + hs_ref[1]
    out_ref[...] = jnp.where(d > 0.0, a / jnp.where(d > 0.0, d, 1.0), 0.0)


def _fin_call(acc_p, hs_r):
    return pl.pallas_call(
        _fin_body,
        grid=(N // _BN,),
        in_specs=[
            pl.BlockSpec((NC, _BN, H), lambda i: (0, i, 0)),
            pl.BlockSpec((NC, _BN, 1), lambda i: (0, i, 0)),
        ],
        out_specs=pl.BlockSpec((_BN, H), lambda i: (i, 0)),
        out_shape=jax.ShapeDtypeStruct((N, H), jnp.float32),
    )(acc_p, hs_r)


@functools.partial(
    pl.kernel,
    out_type=jax.ShapeDtypeStruct((NW, RPW, CB), jnp.float32),
    mesh=plsc.VectorSubcoreMesh(core_axis_name="c", subcore_axis_name="s"),
    scratch_types=[
        pltpu.VMEM((RPW, CB), jnp.float32),
        pltpu.VMEM((RPW, CB), jnp.int32),
        pltpu.VMEM((NP, L), jnp.float32),
        pltpu.VMEM((NP, L), jnp.float32),
        [pltpu.SemaphoreType.DMA] * 4,
    ],
    compiler_params=pltpu.CompilerParams(needs_layout_passes=False, use_tc_tiling_on_sc=False),
)
def _sc_alpha(h_hbm, src_hbm, hsum_hbm, alpha_out,
              h_v, src_v, hs0_v, hs1_v, lsems):
    cid = lax.axis_index("c")
    sid = lax.axis_index("s")
    wid = sid * NC + cid
    ld = [pltpu.async_copy(h_hbm.at[wid], h_v, lsems[0]),
          pltpu.async_copy(src_hbm.at[wid], src_v, lsems[1]),
          pltpu.async_copy(hsum_hbm.at[0], hs0_v, lsems[2]),
          pltpu.async_copy(hsum_hbm.at[1], hs1_v, lsems[3])]
    for d in ld:
        d.wait()

    def _b(r, carry):
        for c in range(CB // 16):
            srcg = src_v[r, pl.ds(c * 16, 16)]
            hs = (plsc.load_gather(hs0_v, [srcg >> 4, srcg & 15])
                  + plsc.load_gather(hs1_v, [srcg >> 4, srcg & 15]))
            h_v[r, pl.ds(c * 16, 16)] = h_v[r, pl.ds(c * 16, 16)] / hs
        return carry
    lax.fori_loop(0, RPW, _b, 0)

    pltpu.sync_copy(h_v, alpha_out.at[wid])


def kernel(x, edge_index, W_fc, W_a, b_a):
    src = edge_index[0].reshape(NW, RPW, CB)
    dst = edge_index[1].reshape(NW, RPW, CB)
    wa2 = W_a.reshape(2, H)
    b2 = jnp.concatenate([b_a, jnp.zeros((1,), jnp.float32)]).reshape(1, 2)

    ht, s = _prep_call(x, W_fc, wa2, b2)
    s1 = s[:, 0]
    s2 = s[:, 1]

    h2, hsum_p = _sc_edge(src, dst, s1, s2)
    acc_p = _sc_agg(src, dst, h2, ht)
    alpha2 = _sc_alpha(h2, src, hsum_p)

    hs_r = hsum_p.reshape(NC, NP * L)[:, :N].reshape(NC, N, 1)
    out = _fin_call(acc_p.reshape(NC, N, H), hs_r)
    return (out, alpha2.reshape(E))


# speedup vs baseline: 1.2016x; 1.0882x over previous
import functools

import jax
import jax.numpy as jnp
from jax import lax
from jax.experimental import pallas as pl
from jax.experimental.pallas import tpu as pltpu
from jax.experimental.pallas import tpu_sc as plsc

N = 10000
E = 320000
D = 128
H = 128
NEG = 0.05

NC, NS, L = 2, 16, 16
NW = NC * NS
CB = 80
ROWS = E // CB
RPW = ROWS // NW
NP = 640
NPT = N // NS
SEG = 25


def _prep_body(x_ref, w_ref, wa_ref, b_ref, ht_ref, s_ref):
    xb = x_ref[...]
    ht = lax.dot_general(xb, w_ref[...], (((1,), (1,)), ((), ())),
                         preferred_element_type=jnp.float32)
    ht_ref[...] = ht.astype(jnp.bfloat16)
    s = lax.dot_general(ht, wa_ref[...], (((1,), (1,)), ((), ())),
                        preferred_element_type=jnp.float32)
    s_ref[...] = s + b_ref[...]


_BN = 1000


def _prep_call(x, w, wa2, b2):
    return pl.pallas_call(
        _prep_body,
        grid=(N // _BN,),
        in_specs=[
            pl.BlockSpec((_BN, D), lambda i: (i, 0)),
            pl.BlockSpec((H, D), lambda i: (0, 0)),
            pl.BlockSpec((2, H), lambda i: (0, 0)),
            pl.BlockSpec((1, 2), lambda i: (0, 0)),
        ],
        out_specs=[
            pl.BlockSpec((_BN, H), lambda i: (i, 0)),
            pl.BlockSpec((_BN, 2), lambda i: (i, 0)),
        ],
        out_shape=[
            jax.ShapeDtypeStruct((N, H), jnp.bfloat16),
            jax.ShapeDtypeStruct((N, 2), jnp.float32),
        ],
    )(x, w, wa2, b2)


@functools.partial(
    pl.kernel,
    out_type=(
        jax.ShapeDtypeStruct((NW, RPW, CB), jnp.float32),
        jax.ShapeDtypeStruct((NC, NP, L), jnp.float32),
    ),
    mesh=plsc.VectorSubcoreMesh(core_axis_name="c", subcore_axis_name="s"),
    scratch_types=[
        pltpu.VMEM((N,), jnp.float32),
        pltpu.VMEM((N,), jnp.float32),
        pltpu.VMEM((RPW, CB), jnp.int32),
        pltpu.VMEM((RPW, CB), jnp.int32),
        pltpu.VMEM((RPW, CB), jnp.float32),
        pltpu.VMEM((NP, L), jnp.float32),
        pltpu.VMEM((NP // 128, 128), jnp.int32),
        pltpu.VMEM_SHARED((NP, L), jnp.float32),
        [pltpu.SemaphoreType.DMA] * 4,
    ],
    compiler_params=pltpu.CompilerParams(needs_layout_passes=False, use_tc_tiling_on_sc=False),
)
def _sc_edge(src_hbm, dst_hbm, s1_hbm, s2_hbm,
             h_out, hsum_out,
             s1_v, s2_v, src_v, dst_v, h_v, hsum_v, ridx_v, hsum_s, lsems):
    cid = lax.axis_index("c")
    sid = lax.axis_index("s")
    wid = sid * NC + cid

    ld = [pltpu.async_copy(s1_hbm, s1_v, lsems[0]),
          pltpu.async_copy(s2_hbm, s2_v, lsems[1]),
          pltpu.async_copy(src_hbm.at[wid], src_v, lsems[2]),
          pltpu.async_copy(dst_hbm.at[wid], dst_v, lsems[3])]

    for k in range(NP // 128):
        for c in range(8):
            ridx_v[k, pl.ds(c * 16, 16)] = (
                lax.iota(jnp.int32, 16) + (k * 128 + c * 16))

    def _z_hsum(r, carry):
        hsum_v[r, :] = jnp.zeros((16,), jnp.float32)
        return carry
    lax.fori_loop(0, NP, _z_hsum, 0)

    @pl.when(sid == 0)
    def _():
        pltpu.sync_copy(hsum_v, hsum_s)

    for d in ld:
        d.wait()

    plsc.subcore_barrier()

    def _p1(r):
        for c in range(CB // 16):
            srcg = src_v[r, pl.ds(c * 16, 16)]
            dstg = dst_v[r, pl.ds(c * 16, 16)]
            v = (plsc.load_gather(s1_v, [srcg])
                 + plsc.load_gather(s2_v, [dstg]))
            v = jnp.where(v >= 0.0, v, NEG * v)
            hg = jnp.exp(v)
            h_v[r, pl.ds(c * 16, 16)] = hg
            plsc.addupdate_scatter(hsum_v, [srcg >> 4, srcg & 15], hg)
    plsc.parallel_loop(0, RPW, unroll=2)(_p1)

    pltpu.sync_copy(h_v, h_out.at[wid])

    for k in range(NP // 128):
        pltpu.sync_copy(hsum_v.at[pl.ds(k * 128, 128)],
                        hsum_s.at[ridx_v.at[k]], add=True)

    plsc.subcore_barrier()

    @pl.when(sid == 0)
    def _():
        pltpu.sync_copy(hsum_s, hsum_out.at[cid])


NB = 4
NFB = 2


@functools.partial(
    pl.kernel,
    out_type=jax.ShapeDtypeStruct((NC, NS, NPT, H), jnp.float32),
    mesh=plsc.VectorSubcoreMesh(core_axis_name="c", subcore_axis_name="s"),
    scratch_types=[
        pltpu.VMEM((SEG, CB), jnp.int32),
        pltpu.VMEM((SEG, CB), jnp.int32),
        pltpu.VMEM((SEG, CB), jnp.float32),
        [pltpu.VMEM((CB, H), jnp.bfloat16)] * NB,
        [pltpu.VMEM((CB, H), jnp.float32)] * NFB,
        [pltpu.SemaphoreType.DMA] * NB,
        [pltpu.SemaphoreType.DMA] * NFB,
        [pltpu.SemaphoreType.DMA] * 3,
        pltpu.VMEM_SHARED((N, H), jnp.float32),
    ],
    compiler_params=pltpu.CompilerParams(needs_layout_passes=False, use_tc_tiling_on_sc=False),
)
def _sc_agg(src_hbm, dst_hbm, h_hbm, ht_hbm, acc_out,
            src_v, dst_v, h_v, rbufs, fbufs, gsems, ssems, lsems, acc_s):
    cid = lax.axis_index("c")
    sid = lax.axis_index("s")
    wid = sid * NC + cid

    def _z_rows(r, carry):
        for c in range(H // 16):
            fbufs[0][r, pl.ds(c * 16, 16)] = jnp.zeros((16,), jnp.float32)
        return carry
    lax.fori_loop(0, CB, _z_rows, 0)

    for k in range(NPT // CB):
        pltpu.sync_copy(fbufs[0], acc_s.at[pl.ds(sid * NPT + k * CB, CB)])
    rem = NPT - (NPT // CB) * CB
    pltpu.sync_copy(fbufs[0].at[pl.ds(0, rem)],
                    acc_s.at[pl.ds(sid * NPT + (NPT // CB) * CB, rem)])

    plsc.subcore_barrier()

    npre = NB - 1
    ev2 = lax.iota(jnp.int32, 16) * 2

    def _seg(g, carry):
        lds = [pltpu.async_copy(src_hbm.at[wid, pl.ds(g * SEG, SEG)],
                                src_v, lsems[0]),
               pltpu.async_copy(dst_hbm.at[wid, pl.ds(g * SEG, SEG)],
                                dst_v, lsems[1]),
               pltpu.async_copy(h_hbm.at[wid, pl.ds(g * SEG, SEG)],
                                h_v, lsems[2])]
        for d in lds:
            d.wait()
        gdescs = {}
        sdescs = {}
        for rr in range(npre):
            gdescs[rr] = pltpu.async_copy(
                ht_hbm.at[dst_v.at[rr]], rbufs[rr % NB], gsems[rr % NB])

        for r in range(SEG):
            nxt = r + npre
            if nxt < SEG:
                gdescs[nxt] = pltpu.async_copy(
                    ht_hbm.at[dst_v.at[nxt]], rbufs[nxt % NB], gsems[nxt % NB])
            gdescs[r].wait()
            if r >= NFB:
                sdescs[r - NFB].wait()
            buf = rbufs[r % NB]
            fb = fbufs[r % NFB]

            def _scale(j, r=r, buf=buf, fb=fb):
                hb = plsc.load_gather(h_v, [jnp.full((16,), r, jnp.int32),
                                            jnp.full((16,), j, jnp.int32)])
                jv = jnp.full((16,), j, jnp.int32)
                for c in range(H // 32):
                    ab = buf[j, pl.ds(c * 32, 32)]
                    a, b = plsc.unpack(ab, format=plsc.PackFormat.INTERLEAVED)
                    plsc.store_scatter(fb, [jv, c * 32 + ev2], a * hb)
                    plsc.store_scatter(fb, [jv, c * 32 + 1 + ev2], b * hb)
            plsc.parallel_loop(0, CB, unroll=4)(_scale)

            sdescs[r] = pltpu.async_copy(fb, acc_s.at[src_v.at[r]],
                                         ssems[r % NFB], add=True)

        for rr in range(max(0, SEG - NFB), SEG):
            sdescs[rr].wait()
        return carry
    lax.fori_loop(0, RPW // SEG, _seg, 0)

    plsc.subcore_barrier()

    pltpu.sync_copy(acc_s.at[pl.ds(sid * NPT, NPT)],
                    acc_out.at[cid, sid])


def _fin_body(acc_ref, hs_ref, out_ref):
    a = acc_ref[0] + acc_ref[1]
    d = hs_ref[0] + hs_ref[1]
    out_ref[...] = jnp.where(d > 0.0, a / jnp.where(d > 0.0, d, 1.0), 0.0)


def _fin_call(acc_p, hs_r):
    return pl.pallas_call(
        _fin_body,
        grid=(N // _BN,),
        in_specs=[
            pl.BlockSpec((NC, _BN, H), lambda i: (0, i, 0)),
            pl.BlockSpec((NC, _BN, 1), lambda i: (0, i, 0)),
        ],
        out_specs=pl.BlockSpec((_BN, H), lambda i: (i, 0)),
        out_shape=jax.ShapeDtypeStruct((N, H), jnp.float32),
    )(acc_p, hs_r)


@functools.partial(
    pl.kernel,
    out_type=jax.ShapeDtypeStruct((NW, RPW, CB), jnp.float32),
    mesh=plsc.VectorSubcoreMesh(core_axis_name="c", subcore_axis_name="s"),
    scratch_types=[
        pltpu.VMEM((RPW, CB), jnp.float32),
        pltpu.VMEM((RPW, CB), jnp.int32),
        pltpu.VMEM((NP, L), jnp.float32),
        pltpu.VMEM((NP, L), jnp.float32),
        [pltpu.SemaphoreType.DMA] * 4,
    ],
    compiler_params=pltpu.CompilerParams(needs_layout_passes=False, use_tc_tiling_on_sc=False),
)
def _sc_alpha(h_hbm, src_hbm, hsum_hbm, alpha_out,
              h_v, src_v, hs0_v, hs1_v, lsems):
    cid = lax.axis_index("c")
    sid = lax.axis_index("s")
    wid = sid * NC + cid
    ld = [pltpu.async_copy(h_hbm.at[wid], h_v, lsems[0]),
          pltpu.async_copy(src_hbm.at[wid], src_v, lsems[1]),
          pltpu.async_copy(hsum_hbm.at[0], hs0_v, lsems[2]),
          pltpu.async_copy(hsum_hbm.at[1], hs1_v, lsems[3])]
    for d in ld:
        d.wait()

    def _b(r):
        for c in range(CB // 16):
            srcg = src_v[r, pl.ds(c * 16, 16)]
            hs = (plsc.load_gather(hs0_v, [srcg >> 4, srcg & 15])
                  + plsc.load_gather(hs1_v, [srcg >> 4, srcg & 15]))
            h_v[r, pl.ds(c * 16, 16)] = h_v[r, pl.ds(c * 16, 16)] / hs
    plsc.parallel_loop(0, RPW, unroll=2)(_b)

    pltpu.sync_copy(h_v, alpha_out.at[wid])


def kernel(x, edge_index, W_fc, W_a, b_a):
    src = edge_index[0].reshape(NW, RPW, CB)
    dst = edge_index[1].reshape(NW, RPW, CB)
    wa2 = W_a.reshape(2, H)
    b2 = jnp.concatenate([b_a, jnp.zeros((1,), jnp.float32)]).reshape(1, 2)

    ht, s = _prep_call(x, W_fc, wa2, b2)
    s1 = s[:, 0]
    s2 = s[:, 1]

    h2, hsum_p = _sc_edge(src, dst, s1, s2)
    acc_p = _sc_agg(src, dst, h2, ht)
    alpha2 = _sc_alpha(h2, src, hsum_p)

    hs_r = hsum_p.reshape(NC, NP * L)[:, :N].reshape(NC, N, 1)
    out = _fin_call(acc_p.reshape(NC, N, H), hs_r)
    return (out, alpha2.reshape(E))
